# baseline (speedup 1.0000x reference)
import functools

import jax
import jax.numpy as jnp
from jax import lax
from jax.experimental import pallas as pl
from jax.experimental.pallas import tpu as pltpu
from jax.experimental.pallas import tpu_sc as plsc

H = 2048
I = 1024
E = 8
R = 128
TOPK = 2
TM = 256

_SC_INFO = plsc.get_sparse_core_info()
_NC = _SC_INFO.num_cores
_NS = _SC_INFO.num_subcores
_NW = _NC * _NS


def _routing_sc(T):
    tpw = T // _NW
    mesh = plsc.VectorSubcoreMesh(core_axis_name="c", subcore_axis_name="s")

    @functools.partial(
        pl.kernel, mesh=mesh,
        out_type=jax.ShapeDtypeStruct((T * E,), jnp.float32),
        scratch_types=[
            pltpu.VMEM((E * tpw,), jnp.float32),
            pltpu.VMEM((E * tpw,), jnp.float32),
        ],
    )
    def routing(lt_hbm, out_hbm, lbuf, obuf):
        wid = lax.axis_index("s") * _NC + lax.axis_index("c")
        base = wid * tpw
        for e in range(E):
            pltpu.sync_copy(lt_hbm.at[e, pl.ds(base, tpw)],
                            lbuf.at[pl.ds(e * tpw, tpw)])
        for c in range(tpw // 16):
            vs = [lbuf[pl.ds(e * tpw + c * 16, 16)] for e in range(E)]
            mx = vs[0]
            for e in range(1, E):
                mx = jnp.maximum(mx, vs[e])
            ps = [jnp.exp(v - mx) for v in vs]
            s = ps[0]
            for e in range(1, E):
                s = s + ps[e]
            ps = [p / s for p in ps]
            one = jnp.full((16,), 1.0, jnp.float32)
            zero = jnp.full((16,), 0.0, jnp.float32)
            m1 = ps[0]
            for e in range(1, E):
                m1 = jnp.maximum(m1, ps[e])
            seen = zero
            sel1 = []
            for e in range(E):
                is1 = jnp.where(ps[e] == m1, one, zero) * (one - seen)
                seen = seen + is1
                sel1.append(is1)
            qs = [ps[e] - 2.0 * sel1[e] for e in range(E)]
            m2 = qs[0]
            for e in range(1, E):
                m2 = jnp.maximum(m2, qs[e])
            seen2 = zero
            for e in range(E):
                is2 = jnp.where(qs[e] == m2, one, zero) * (one - seen2)
                seen2 = seen2 + is2
                w = ps[e] * (sel1[e] + is2)
                obuf[pl.ds(e * tpw + c * 16, 16)] = w
        pltpu.sync_copy(obuf, out_hbm.at[pl.ds(base * E, tpw * E)])

    return routing


def _mm(a, b):
    return jax.lax.dot_general(
        a, b, (((1,), (1,)), ((), ())), preferred_element_type=jnp.float32
    )


def _moe_body(xb_ref, lg_ref, wd_ref, bgu_ref, allA_ref, gB_ref, uB_ref,
              dA_ref, bd_ref, dB_ref, out_ref, logits_ref):
    logits_ref[...] = lg_ref[...]
    wdense = wd_ref[...]

    bf = jnp.bfloat16
    xb = xb_ref[...].astype(bf)
    GU = _mm(xb, bgu_ref[...])
    AA = _mm(xb, allA_ref[...])
    G0 = GU[:, :I]
    U0 = GU[:, I:]

    Hw = jnp.zeros((TM, I), jnp.float32)
    wads = []
    for e in range(E):
        Ag = AA[:, e * 2 * R:e * 2 * R + R].astype(bf)
        Au = AA[:, e * 2 * R + R:(e + 1) * 2 * R].astype(bf)
        g = G0 + _mm(Ag, gB_ref[e])
        u = U0 + _mm(Au, uB_ref[e])
        h = g * jax.nn.sigmoid(g) * u
        we = wdense[:, e:e + 1]
        Hw = Hw + h * we
        Ad = _mm(h.astype(bf), dA_ref[e])
        wads.append(Ad * we)
    WAd = jnp.concatenate(wads, axis=1).astype(bf)
    d = _mm(Hw.astype(bf), bd_ref[...]) + _mm(WAd, dB_ref[...])
    out_ref[...] = d


def kernel(hidden_states, gate_w, base_gate, base_up, base_down,
           gate_A, gate_B, up_A, up_B, down_A, down_B):
    b, s_len, h = hidden_states.shape
    T = b * s_len
    x = hidden_states.reshape(T, h)
    router_logits = x @ gate_w.T
    wflat = _routing_sc(T)(router_logits.T)
    wdense = wflat.reshape(_NW, E, T // _NW).transpose(0, 2, 1).reshape(T, E)
    bf = jnp.bfloat16
    bgu = jnp.concatenate([base_gate, base_up], axis=0).astype(bf)
    allA = jnp.concatenate([gate_A, up_A], axis=1).reshape(2 * R * E, H).astype(bf)
    dBcat = jnp.transpose(down_B, (1, 0, 2)).reshape(H, E * R).astype(bf)
    full = lambda shape: pl.BlockSpec(shape, lambda i: (0,) * len(shape))
    final, logits = pl.pallas_call(
        _moe_body,
        grid=(T // TM,),
        in_specs=[
            pl.BlockSpec((TM, H), lambda i: (i, 0)),
            pl.BlockSpec((TM, E), lambda i: (i, 0)),
            pl.BlockSpec((TM, E), lambda i: (i, 0)),
            full((2 * I, H)),
            full((2 * R * E, H)),
            full((E, I, R)),
            full((E, I, R)),
            full((E, R, I)),
            full((H, I)),
            full((H, E * R)),
        ],
        out_specs=[
            pl.BlockSpec((TM, H), lambda i: (i, 0)),
            pl.BlockSpec((TM, E), lambda i: (i, 0)),
        ],
        out_shape=[
            jax.ShapeDtypeStruct((T, H), jnp.float32),
            jax.ShapeDtypeStruct((T, E), jnp.float32),
        ],
        compiler_params=pltpu.CompilerParams(
            dimension_semantics=("arbitrary",),
        ),
    )(x, router_logits, wdense, bgu, allA, gate_B.astype(bf),
      up_B.astype(bf), down_A.astype(bf), base_down.astype(bf), dBcat)
    return final.reshape(b, s_len, h), logits


# speedup vs baseline: 1.1334x; 1.1334x over previous
import jax
import jax.numpy as jnp
from jax.experimental import pallas as pl
from jax.experimental.pallas import tpu as pltpu

H = 2048
I = 1024
E = 8
R = 128
TOPK = 2
TM = 256


def _mm(a, b):
    return jax.lax.dot_general(
        a, b, (((1,), (1,)), ((), ())), preferred_element_type=jnp.float32
    )


def _moe_body(xb_ref, lg_ref, bgu_ref, allA_ref, gB_ref, uB_ref, dA_ref,
              bd_ref, dB_ref, out_ref, logits_ref):
    logits = lg_ref[...]
    logits_ref[...] = logits

    m = jnp.max(logits, axis=1, keepdims=True)
    p = jnp.exp(logits - m)
    p = p / jnp.sum(p, axis=1, keepdims=True)
    iot = jax.lax.broadcasted_iota(jnp.int32, (TM, E), 1)
    m1 = jnp.max(p, axis=1, keepdims=True)
    a1 = jnp.min(jnp.where(p == m1, iot, E), axis=1, keepdims=True)
    mask1 = iot == a1
    p2 = jnp.where(mask1, -jnp.inf, p)
    m2 = jnp.max(p2, axis=1, keepdims=True)
    a2 = jnp.min(jnp.where(p2 == m2, iot, E), axis=1, keepdims=True)
    wdense = jnp.where(mask1 | (iot == a2), p, 0.0)

    bf = jnp.bfloat16
    xb = xb_ref[...].astype(bf)
    GU = _mm(xb, bgu_ref[...])
    AA = _mm(xb, allA_ref[...])
    G0 = GU[:, :I]
    U0 = GU[:, I:]

    Hw = jnp.zeros((TM, I), jnp.float32)
    wads = []
    for e in range(E):
        Ag = AA[:, e * 2 * R:e * 2 * R + R].astype(bf)
        Au = AA[:, e * 2 * R + R:(e + 1) * 2 * R].astype(bf)
        g = G0 + _mm(Ag, gB_ref[e])
        u = U0 + _mm(Au, uB_ref[e])
        h = g * jax.nn.sigmoid(g) * u
        we = wdense[:, e:e + 1]
        Hw = Hw + h * we
        Ad = _mm(h.astype(bf), dA_ref[e])
        wads.append(Ad * we)
    WAd = jnp.concatenate(wads, axis=1).astype(bf)
    d = _mm(Hw.astype(bf), bd_ref[...]) + _mm(WAd, dB_ref[...])
    out_ref[...] = d


def kernel(hidden_states, gate_w, base_gate, base_up, base_down,
           gate_A, gate_B, up_A, up_B, down_A, down_B):
    b, s_len, h = hidden_states.shape
    T = b * s_len
    x = hidden_states.reshape(T, h)
    router_logits = x @ gate_w.T
    bf = jnp.bfloat16
    bgu = jnp.concatenate([base_gate, base_up], axis=0).astype(bf)
    allA = jnp.concatenate([gate_A, up_A], axis=1).reshape(2 * R * E, H).astype(bf)
    dBcat = jnp.transpose(down_B, (1, 0, 2)).reshape(H, E * R).astype(bf)
    full = lambda shape: pl.BlockSpec(shape, lambda i: (0,) * len(shape))
    final, logits = pl.pallas_call(
        _moe_body,
        grid=(T // TM,),
        in_specs=[
            pl.BlockSpec((TM, H), lambda i: (i, 0)),
            pl.BlockSpec((TM, E), lambda i: (i, 0)),
            full((2 * I, H)),
            full((2 * R * E, H)),
            full((E, I, R)),
            full((E, I, R)),
            full((E, R, I)),
            full((H, I)),
            full((H, E * R)),
        ],
        out_specs=[
            pl.BlockSpec((TM, H), lambda i: (i, 0)),
            pl.BlockSpec((TM, E), lambda i: (i, 0)),
        ],
        out_shape=[
            jax.ShapeDtypeStruct((T, H), jnp.float32),
            jax.ShapeDtypeStruct((T, E), jnp.float32),
        ],
        compiler_params=pltpu.CompilerParams(
            dimension_semantics=("arbitrary",),
        ),
    )(x, router_logits, bgu, allA, gate_B.astype(bf),
      up_B.astype(bf), down_A.astype(bf), base_down.astype(bf), dBcat)
    return final.reshape(b, s_len, h), logits
